# baseline (speedup 1.0000x reference)
import functools

import jax
import jax.numpy as jnp
from jax import lax
from jax.experimental import pallas as pl
from jax.experimental.pallas import tpu as pltpu
from jax.experimental.pallas import tpu_sc as plsc

N = 10000
E = 320000
D = 128
DO = 64
NG = 64
NSC = 16
CH = 64
CPT = 320
EPT = CPT * CH
EP = EPT * NSC
SROWS = 10240
RPT = SROWS // NSC
NB = 10
BLK = SROWS // NB
NBUF = 4
IG = 64
NGRP = CPT // IG

_f32 = jnp.float32


def _sc_mesh():
    return plsc.VectorSubcoreMesh(core_axis_name="c", subcore_axis_name="s")


def _degree_kernel(col_hbm, ones_hbm, zr_hbm, out_hbm, acc, cidx, ones_v):
    c = lax.axis_index("c")
    s = lax.axis_index("s")
    pltpu.sync_copy(ones_hbm, ones_v)
    pltpu.sync_copy(zr_hbm, acc.at[pl.ds(s * RPT, RPT)])
    plsc.subcore_barrier()

    def group(g, _):
        pltpu.sync_copy(col_hbm.at[c, s, g], cidx)

        def body(j, _):
            pltpu.sync_copy(ones_v, acc.at[cidx.at[j]], add=True)
            return 0

        lax.fori_loop(0, IG, body, 0)
        return 0

    lax.fori_loop(0, NGRP, group, 0)
    plsc.subcore_barrier()
    r0 = s * RPT
    pltpu.sync_copy(acc.at[pl.ds(r0, RPT)], out_hbm.at[c].at[pl.ds(r0, RPT)])


def _sc_degree(colg, ones128, zr128):
    k = functools.partial(
        pl.kernel,
        out_type=jax.ShapeDtypeStruct((2, SROWS, D), _f32),
        mesh=_sc_mesh(),
        scratch_types=[
            pltpu.VMEM_SHARED((SROWS, D), _f32),
            pltpu.VMEM((IG, CH), jnp.int32),
            pltpu.VMEM((CH, D), _f32),
        ],
    )(_degree_kernel)
    return k(colg, ones128, zr128)


def _conv_kernel(h_hbm, row_hbm, col_hbm, zr_hbm, out_hbm, acc, ridx, cidx,
                 gbuf, gsem, ssem):
    c = lax.axis_index("c")
    s = lax.axis_index("s")
    pltpu.sync_copy(zr_hbm, acc.at[pl.ds(s * RPT, RPT)])
    plsc.subcore_barrier()

    def group(g, _):
        pltpu.sync_copy(row_hbm.at[c, s, g], ridx)
        pltpu.sync_copy(col_hbm.at[c, s, g], cidx)
        for b in range(3):
            pltpu.async_copy(h_hbm.at[ridx.at[b]], gbuf.at[b], gsem.at[b])

        def body(jj, _):
            j0 = jj * NBUF
            for u in range(NBUF):
                j = j0 + u
                bj = u
                bn = (u + 3) % NBUF
                if u == 0:
                    @pl.when((g > 0) | (jj > 0))
                    def _():
                        pltpu.make_async_copy(gbuf.at[bn], acc.at[cidx.at[j]],
                                              ssem.at[bn]).wait()
                else:
                    pltpu.make_async_copy(gbuf.at[bn], acc.at[cidx.at[j]],
                                          ssem.at[bn]).wait()

                @pl.when(j + 3 < IG)
                def _():
                    pltpu.async_copy(h_hbm.at[ridx.at[j + 3]], gbuf.at[bn],
                                     gsem.at[bn])

                pltpu.make_async_copy(h_hbm.at[ridx.at[j]], gbuf.at[bj],
                                      gsem.at[bj]).wait()
                pltpu.async_copy(gbuf.at[bj], acc.at[cidx.at[j]], ssem.at[bj],
                                 add=True)
            return 0

        lax.fori_loop(0, IG // NBUF, body, 0)
        return 0

    lax.fori_loop(0, NGRP, group, 0)
    pltpu.make_async_copy(gbuf.at[3], acc.at[cidx.at[IG - 1]],
                          ssem.at[3]).wait()
    plsc.subcore_barrier()
    r0 = s * RPT
    pltpu.sync_copy(acc.at[pl.ds(r0, RPT)], out_hbm.at[c].at[pl.ds(r0, RPT)])


def _sc_conv(h_flat, rowg, colg, zr128):
    k = functools.partial(
        pl.kernel,
        out_type=jax.ShapeDtypeStruct((2, SROWS, D), _f32),
        mesh=_sc_mesh(),
        scratch_types=[
            pltpu.VMEM_SHARED((SROWS, D), _f32),
            pltpu.VMEM((IG, CH), jnp.int32),
            pltpu.VMEM((IG, CH), jnp.int32),
            pltpu.VMEM((NBUF, CH, D), _f32),
            pltpu.SemaphoreType.DMA((NBUF,)),
            pltpu.SemaphoreType.DMA((NBUF,)),
        ],
    )(_conv_kernel)
    return k(h_flat, rowg, colg, zr128)


def _dis(degw_ref):
    return lax.rsqrt(degw_ref[0, :, 0:1] + 1.0)


def _k1a_body(x_ref, w1_ref, out_ref):
    out_ref[0] = jnp.dot(x_ref[0], w1_ref[...], preferred_element_type=_f32)


def _tc_k1a(xs, w1):
    return pl.pallas_call(
        _k1a_body,
        grid=(2, NB),
        in_specs=[
            pl.BlockSpec((1, BLK, D), lambda i, j: (i, j, 0)),
            pl.BlockSpec((D, D), lambda i, j: (0, 0)),
        ],
        out_specs=pl.BlockSpec((1, BLK, D), lambda i, j: (i, j, 0)),
        out_shape=jax.ShapeDtypeStruct((2, SROWS, D), _f32),
    )(xs, w1)


def _k1b_body(h_ref, degw_ref, out_ref):
    out_ref[0] = _dis(degw_ref) * h_ref[0]


def _tc_k1b(h, degw):
    return pl.pallas_call(
        _k1b_body,
        grid=(2, NB),
        in_specs=[
            pl.BlockSpec((1, BLK, D), lambda i, j: (i, j, 0)),
            pl.BlockSpec((1, BLK, D), lambda i, j: (i, j, 0)),
        ],
        out_specs=pl.BlockSpec((1, BLK, D), lambda i, j: (i, j, 0)),
        out_shape=jax.ShapeDtypeStruct((2, SROWS, D), _f32),
    )(h, degw)


def _k3_body(s_ref, h_ref, degw_ref, b1_ref, w2_ref, out_ref):
    dis = _dis(degw_ref)
    t = jnp.maximum(dis * (s_ref[0] + h_ref[0]) + b1_ref[...], 0.0)
    out_ref[0] = dis * jnp.dot(t, w2_ref[...], preferred_element_type=_f32)


def _tc_k3(s1, h1, degw, b1r, w2):
    return pl.pallas_call(
        _k3_body,
        grid=(2, NB),
        in_specs=[
            pl.BlockSpec((1, BLK, D), lambda i, j: (i, j, 0)),
            pl.BlockSpec((1, BLK, D), lambda i, j: (i, j, 0)),
            pl.BlockSpec((1, BLK, D), lambda i, j: (i, j, 0)),
            pl.BlockSpec((1, D), lambda i, j: (0, 0)),
            pl.BlockSpec((D, D), lambda i, j: (0, 0)),
        ],
        out_specs=pl.BlockSpec((1, BLK, D), lambda i, j: (i, j, 0)),
        out_shape=jax.ShapeDtypeStruct((2, SROWS, D), _f32),
    )(s1, h1, degw, b1r, w2)


def _k5_body(s_ref, h_ref, degw_ref, b2_ref, bf_ref, mw1_ref, mb1_ref,
             mw2_ref, mb2_ref, out_ref, ps0, cs0, ps1, cs1):
    i = pl.program_id(0)
    j = pl.program_id(1)
    dis = _dis(degw_ref)
    u = jnp.maximum(dis * (s_ref[0] + h_ref[0]) + b2_ref[...], 0.0)
    gids = lax.broadcasted_iota(jnp.int32, (BLK, NG), 1).astype(_f32)
    oh = (bf_ref[0] == gids).astype(_f32)
    ps = lax.dot_general(oh, u, (((0,), (0,)), ((), ())),
                         preferred_element_type=_f32)
    cs = jnp.broadcast_to(jnp.sum(oh, axis=0)[:, None], (NG, D))

    def accum(psr, csr):
        @pl.when(j == 0)
        def _():
            psr[...] = ps
            csr[...] = cs

        @pl.when(j != 0)
        def _():
            psr[...] += ps
            csr[...] += cs

    @pl.when(i == 0)
    def _():
        accum(ps0, cs0)

    @pl.when(i == 1)
    def _():
        accum(ps1, cs1)

    @pl.when((i == 1) & (j == NB - 1))
    def _():
        p0 = ps0[...] / jnp.maximum(cs0[...], 1.0)
        p1 = ps1[...] / jnp.maximum(cs1[...], 1.0)
        hp = p0 - p1
        z = jnp.maximum(
            jnp.dot(hp, mw1_ref[...], preferred_element_type=_f32)
            + mb1_ref[...], 0.0)
        out_ref[...] = (jnp.dot(z, mw2_ref[...], preferred_element_type=_f32)
                        + mb2_ref[...])


def _tc_k5(s2, h2, degw, b2r, batchf, mw1, mb1r, mw2, mb2r):
    return pl.pallas_call(
        _k5_body,
        grid=(2, NB),
        in_specs=[
            pl.BlockSpec((1, BLK, D), lambda i, j: (i, j, 0)),
            pl.BlockSpec((1, BLK, D), lambda i, j: (i, j, 0)),
            pl.BlockSpec((1, BLK, D), lambda i, j: (i, j, 0)),
            pl.BlockSpec((1, D), lambda i, j: (0, 0)),
            pl.BlockSpec((1, BLK, 1), lambda i, j: (i, j, 0)),
            pl.BlockSpec((D, D), lambda i, j: (0, 0)),
            pl.BlockSpec((1, D), lambda i, j: (0, 0)),
            pl.BlockSpec((D, DO), lambda i, j: (0, 0)),
            pl.BlockSpec((1, DO), lambda i, j: (0, 0)),
        ],
        out_specs=pl.BlockSpec((NG, DO), lambda i, j: (0, 0)),
        out_shape=jax.ShapeDtypeStruct((NG, DO), _f32),
        scratch_shapes=[
            pltpu.VMEM((NG, D), _f32),
            pltpu.VMEM((NG, D), _f32),
            pltpu.VMEM((NG, D), _f32),
            pltpu.VMEM((NG, D), _f32),
        ],
    )(s2, h2, degw, b2r, batchf, mw1, mb1r, mw2, mb2r)


def kernel(x1, edge_index1, batch1, x2, edge_index2, batch2,
           W1, b1, W2, b2, mW1, mb1, mW2, mb2):
    xs = jnp.pad(jnp.stack([x1, x2]), ((0, 0), (0, SROWS - N), (0, 0)))

    def prep(ei, c):
        rowp = jnp.concatenate(
            [ei[0] + c * SROWS, jnp.full((EP - E,), c * SROWS, jnp.int32)])
        colp = jnp.concatenate(
            [ei[1], jnp.full((EP - E,), N, jnp.int32)])
        return rowp, colp

    r1, c1 = prep(edge_index1, 0)
    r2, c2 = prep(edge_index2, 1)
    rowg = jnp.stack([r1, r2]).reshape(2, NSC, NGRP, IG, CH)
    colg = jnp.stack([c1, c2]).reshape(2, NSC, NGRP, IG, CH)

    ones128 = jnp.ones((CH, D), _f32)
    zr128 = jnp.zeros((RPT, D), _f32)
    batchf = jnp.pad(jnp.stack([batch1, batch2]), ((0, 0), (0, SROWS - N)),
                     constant_values=-1).astype(_f32)[..., None]
    b1r = b1.reshape(1, D)
    b2r = b2.reshape(1, D)
    mb1r = mb1.reshape(1, D)
    mb2r = mb2.reshape(1, DO)

    hraw = _tc_k1a(xs, W1)
    degw = _sc_degree(colg, ones128, zr128)
    h1 = _tc_k1b(hraw, degw)
    s1 = _sc_conv(h1.reshape(2 * SROWS, D), rowg, colg, zr128)
    h2 = _tc_k3(s1, h1, degw, b1r, W2)
    s2 = _sc_conv(h2.reshape(2 * SROWS, D), rowg, colg, zr128)
    return _tc_k5(s2, h2, degw, b2r, batchf, mW1, mb1r, mW2, mb2r)


# speedup vs baseline: 1.1542x; 1.1542x over previous
import functools

import jax
import jax.numpy as jnp
from jax import lax
from jax.experimental import pallas as pl
from jax.experimental.pallas import tpu as pltpu
from jax.experimental.pallas import tpu_sc as plsc

N = 10000
E = 320000
D = 128
DO = 64
NG = 64
NSC = 16
CH = 64
CPT = 320
EPT = CPT * CH
EP = EPT * NSC
SROWS = 10112
RPT = SROWS // NSC
NB = 8
BLK = SROWS // NB
NBUF = 5
IG = 20
NGRP = CPT // IG
DW = D

_f32 = jnp.float32


def _sc_mesh():
    return plsc.VectorSubcoreMesh(core_axis_name="c", subcore_axis_name="s")


def _degree_kernel(col_hbm, ones_hbm, zr_hbm, out_hbm, acc, cidx, ones_v):
    c = lax.axis_index("c")
    s = lax.axis_index("s")
    pltpu.sync_copy(ones_hbm, ones_v)
    pltpu.sync_copy(zr_hbm, acc.at[pl.ds(s * RPT, RPT)])
    plsc.subcore_barrier()

    def group(g, _):
        pltpu.sync_copy(col_hbm.at[c, s, g], cidx)

        def body(j, _):
            pltpu.sync_copy(ones_v, acc.at[cidx.at[j]], add=True)
            return 0

        lax.fori_loop(0, IG, body, 0)
        return 0

    lax.fori_loop(0, NGRP, group, 0)
    plsc.subcore_barrier()
    r0 = s * RPT
    pltpu.sync_copy(acc.at[pl.ds(r0, RPT)], out_hbm.at[c].at[pl.ds(r0, RPT)])


def _sc_degree(colg, onesw, zrw):
    k = functools.partial(
        pl.kernel,
        out_type=jax.ShapeDtypeStruct((2, SROWS, DW), _f32),
        mesh=_sc_mesh(),
        scratch_types=[
            pltpu.VMEM_SHARED((SROWS, DW), _f32),
            pltpu.VMEM((IG, CH), jnp.int32),
            pltpu.VMEM((CH, DW), _f32),
        ],
    )(_degree_kernel)
    return k(colg, onesw, zrw)


def _conv_kernel(h_hbm, row_hbm, col_hbm, zr_hbm, out_hbm, acc, ridx, cidx,
                 gbuf, gsem, ssem):
    c = lax.axis_index("c")
    s = lax.axis_index("s")
    pltpu.sync_copy(zr_hbm, acc.at[pl.ds(s * RPT, RPT)])
    plsc.subcore_barrier()

    def group(g, _):
        pltpu.sync_copy(row_hbm.at[c, s, g], ridx)
        pltpu.sync_copy(col_hbm.at[c, s, g], cidx)
        for b in range(4):
            pltpu.async_copy(h_hbm.at[ridx.at[b]], gbuf.at[b], gsem.at[b])

        def body(jj, _):
            j0 = jj * NBUF
            for u in range(NBUF):
                j = j0 + u
                bj = u
                bn = (u + 4) % NBUF
                if u == 0:
                    @pl.when((g > 0) | (jj > 0))
                    def _():
                        pltpu.make_async_copy(gbuf.at[bn], acc.at[cidx.at[j]],
                                              ssem.at[bn]).wait()
                else:
                    pltpu.make_async_copy(gbuf.at[bn], acc.at[cidx.at[j]],
                                          ssem.at[bn]).wait()

                @pl.when(j + 4 < IG)
                def _():
                    pltpu.async_copy(h_hbm.at[ridx.at[j + 4]], gbuf.at[bn],
                                     gsem.at[bn])

                pltpu.make_async_copy(h_hbm.at[ridx.at[j]], gbuf.at[bj],
                                      gsem.at[bj]).wait()
                pltpu.async_copy(gbuf.at[bj], acc.at[cidx.at[j]], ssem.at[bj],
                                 add=True)
            return 0

        lax.fori_loop(0, IG // NBUF, body, 0)
        return 0

    lax.fori_loop(0, NGRP, group, 0)
    pltpu.make_async_copy(gbuf.at[4], acc.at[cidx.at[IG - 1]],
                          ssem.at[4]).wait()
    plsc.subcore_barrier()
    r0 = s * RPT
    pltpu.sync_copy(acc.at[pl.ds(r0, RPT)], out_hbm.at[c].at[pl.ds(r0, RPT)])


def _sc_conv(h_flat, rowg, colg, zr128):
    k = functools.partial(
        pl.kernel,
        out_type=jax.ShapeDtypeStruct((2, SROWS, D), _f32),
        mesh=_sc_mesh(),
        scratch_types=[
            pltpu.VMEM_SHARED((SROWS, D), _f32),
            pltpu.VMEM((IG, CH), jnp.int32),
            pltpu.VMEM((IG, CH), jnp.int32),
            pltpu.VMEM((NBUF, CH, D), _f32),
            pltpu.SemaphoreType.DMA((NBUF,)),
            pltpu.SemaphoreType.DMA((NBUF,)),
        ],
    )(_conv_kernel)
    return k(h_flat, rowg, colg, zr128)


def _dis(degw_ref):
    return lax.rsqrt(degw_ref[0, :, 0:1] + 1.0)


def _k1_body(x_ref, degw_ref, w1_ref, out_ref):
    h = jnp.dot(x_ref[0], w1_ref[...], preferred_element_type=_f32)
    out_ref[0] = _dis(degw_ref) * h


def _tc_k1(xs, degw, w1):
    return pl.pallas_call(
        _k1_body,
        grid=(2, NB),
        in_specs=[
            pl.BlockSpec((1, BLK, D), lambda i, j: (i, j, 0)),
            pl.BlockSpec((1, BLK, DW), lambda i, j: (i, j, 0)),
            pl.BlockSpec((D, D), lambda i, j: (0, 0)),
        ],
        out_specs=pl.BlockSpec((1, BLK, D), lambda i, j: (i, j, 0)),
        out_shape=jax.ShapeDtypeStruct((2, SROWS, D), _f32),
    )(xs, degw, w1)


def _k3_body(s_ref, h_ref, degw_ref, b1_ref, w2_ref, out_ref):
    dis = _dis(degw_ref)
    t = jnp.maximum(dis * (s_ref[0] + h_ref[0]) + b1_ref[...], 0.0)
    out_ref[0] = dis * jnp.dot(t, w2_ref[...], preferred_element_type=_f32)


def _tc_k3(s1, h1, degw, b1r, w2):
    return pl.pallas_call(
        _k3_body,
        grid=(2, NB),
        in_specs=[
            pl.BlockSpec((1, BLK, D), lambda i, j: (i, j, 0)),
            pl.BlockSpec((1, BLK, D), lambda i, j: (i, j, 0)),
            pl.BlockSpec((1, BLK, DW), lambda i, j: (i, j, 0)),
            pl.BlockSpec((1, D), lambda i, j: (0, 0)),
            pl.BlockSpec((D, D), lambda i, j: (0, 0)),
        ],
        out_specs=pl.BlockSpec((1, BLK, D), lambda i, j: (i, j, 0)),
        out_shape=jax.ShapeDtypeStruct((2, SROWS, D), _f32),
    )(s1, h1, degw, b1r, w2)


def _k5_body(s_ref, h_ref, degw_ref, b2_ref, bf_ref, mw1_ref, mb1_ref,
             mw2_ref, mb2_ref, out_ref, ps0, cs0, ps1, cs1):
    i = pl.program_id(0)
    j = pl.program_id(1)
    dis = _dis(degw_ref)
    u = jnp.maximum(dis * (s_ref[0] + h_ref[0]) + b2_ref[...], 0.0)
    gids = lax.broadcasted_iota(jnp.int32, (BLK, NG), 1).astype(_f32)
    oh = (bf_ref[0] == gids).astype(_f32)
    ps = lax.dot_general(oh, u, (((0,), (0,)), ((), ())),
                         preferred_element_type=_f32)
    cs = jnp.broadcast_to(jnp.sum(oh, axis=0)[:, None], (NG, D))

    def accum(psr, csr):
        @pl.when(j == 0)
        def _():
            psr[...] = ps
            csr[...] = cs

        @pl.when(j != 0)
        def _():
            psr[...] += ps
            csr[...] += cs

    @pl.when(i == 0)
    def _():
        accum(ps0, cs0)

    @pl.when(i == 1)
    def _():
        accum(ps1, cs1)

    @pl.when((i == 1) & (j == NB - 1))
    def _():
        p0 = ps0[...] / jnp.maximum(cs0[...], 1.0)
        p1 = ps1[...] / jnp.maximum(cs1[...], 1.0)
        hp = p0 - p1
        z = jnp.maximum(
            jnp.dot(hp, mw1_ref[...], preferred_element_type=_f32)
            + mb1_ref[...], 0.0)
        out_ref[...] = (jnp.dot(z, mw2_ref[...], preferred_element_type=_f32)
                        + mb2_ref[...])


def _tc_k5(s2, h2, degw, b2r, batchf, mw1, mb1r, mw2, mb2r):
    return pl.pallas_call(
        _k5_body,
        grid=(2, NB),
        in_specs=[
            pl.BlockSpec((1, BLK, D), lambda i, j: (i, j, 0)),
            pl.BlockSpec((1, BLK, D), lambda i, j: (i, j, 0)),
            pl.BlockSpec((1, BLK, DW), lambda i, j: (i, j, 0)),
            pl.BlockSpec((1, D), lambda i, j: (0, 0)),
            pl.BlockSpec((1, BLK, 1), lambda i, j: (i, j, 0)),
            pl.BlockSpec((D, D), lambda i, j: (0, 0)),
            pl.BlockSpec((1, D), lambda i, j: (0, 0)),
            pl.BlockSpec((D, DO), lambda i, j: (0, 0)),
            pl.BlockSpec((1, DO), lambda i, j: (0, 0)),
        ],
        out_specs=pl.BlockSpec((NG, DO), lambda i, j: (0, 0)),
        out_shape=jax.ShapeDtypeStruct((NG, DO), _f32),
        scratch_shapes=[
            pltpu.VMEM((NG, D), _f32),
            pltpu.VMEM((NG, D), _f32),
            pltpu.VMEM((NG, D), _f32),
            pltpu.VMEM((NG, D), _f32),
        ],
    )(s2, h2, degw, b2r, batchf, mw1, mb1r, mw2, mb2r)


def kernel(x1, edge_index1, batch1, x2, edge_index2, batch2,
           W1, b1, W2, b2, mW1, mb1, mW2, mb2):
    xs = jnp.pad(jnp.stack([x1, x2]), ((0, 0), (0, SROWS - N), (0, 0)))

    def prep(ei, c):
        rowp = jnp.concatenate(
            [ei[0] + c * SROWS, jnp.full((EP - E,), c * SROWS, jnp.int32)])
        colp = jnp.concatenate(
            [ei[1], jnp.full((EP - E,), N, jnp.int32)])
        return rowp, colp

    r1, c1 = prep(edge_index1, 0)
    r2, c2 = prep(edge_index2, 1)
    rowg = jnp.stack([r1, r2]).reshape(2, NSC, NGRP, IG, CH)
    colg = jnp.stack([c1, c2]).reshape(2, NSC, NGRP, IG, CH)

    onesw = jnp.ones((CH, DW), _f32)
    zr128 = jnp.zeros((RPT, D), _f32)
    zrw = zr128
    batchf = jnp.pad(jnp.stack([batch1, batch2]), ((0, 0), (0, SROWS - N)),
                     constant_values=-1).astype(_f32)[..., None]
    b1r = b1.reshape(1, D)
    b2r = b2.reshape(1, D)
    mb1r = mb1.reshape(1, D)
    mb2r = mb2.reshape(1, DO)

    degw = _sc_degree(colg, onesw, zrw)
    h1 = _tc_k1(xs, degw, W1)
    s1 = _sc_conv(h1.reshape(2 * SROWS, D), rowg, colg, zr128)
    h2 = _tc_k3(s1, h1, degw, b1r, W2)
    s2 = _sc_conv(h2.reshape(2 * SROWS, D), rowg, colg, zr128)
    return _tc_k5(s2, h2, degw, b2r, batchf, mW1, mb1r, mW2, mb2r)
